# baseline (speedup 1.0000x reference)
import functools
import math

import jax
import jax.numpy as jnp
from jax import lax
from jax.experimental import pallas as pl
from jax.experimental.pallas import tpu as pltpu
from jax.experimental.pallas import tpu_sc as plsc

B = 8
N = 65536
C2 = 32
H_T = 256
W_F = 256
G = H_T * W_F
Q = B * N
NFREQ = 8
CHR_EMB = 8
EP = 16
T1 = 2048
T2 = 2048
NC = 2
NS = 16
NW = NC * NS
QPW = Q // NW
CCH = 1024
SUB = 128


def _stage1(zp, Wbig, bias128):
    TR = T1 // 8

    def body(z_ref, w_ref, b_ref, y_ref):
        y = jnp.dot(z_ref[...], w_ref[...],
                    preferred_element_type=jnp.float32)
        y_ref[...] = y + b_ref[...]

    return pl.pallas_call(
        body,
        grid=(B * G // T1,),
        in_specs=[
            pl.BlockSpec((TR, 8 * C2), lambda g: (g, 0)),
            pl.BlockSpec((8 * C2, 8 * EP), lambda g: (0, 0)),
            pl.BlockSpec((1, 8 * EP), lambda g: (0, 0)),
        ],
        out_specs=pl.BlockSpec((TR, 8 * EP), lambda g: (g, 0)),
        out_shape=jax.ShapeDtypeStruct((B * G // 8, 8 * EP), jnp.float32),
    )(zp, Wbig, bias128)


def _stage2(chrf, start, length, bu, bv, Achr, Wpos, Wj2a):
    nb = N // T2

    def body(c_ref, s_ref, l_ref, bu_ref, bv_ref, a_ref, wp_ref, w2_ref,
             idx_ref, w_ref):
        pid = pl.program_id(0)
        boff = (pid // nb) * G
        ci = c_ref[...].astype(jnp.int32)
        oh = (lax.broadcasted_iota(jnp.int32, (24, 1), 0) == ci
              ).astype(jnp.float32)
        fr = jnp.exp2(lax.broadcasted_iota(jnp.int32, (NFREQ, 1), 0)
                      .astype(jnp.float32))
        tp = 2.0 * math.pi
        s = tp * (fr * s_ref[...])
        l = tp * (fr * l_ref[...])
        pos = jnp.concatenate(
            [jnp.sin(s), jnp.cos(s), jnp.sin(l), jnp.cos(l)], axis=0)
        h = (jnp.dot(a_ref[...], oh, preferred_element_type=jnp.float32)
             + jnp.dot(wp_ref[...], pos, preferred_element_type=jnp.float32))
        h = jnp.where(h >= 0, h, 0.2 * h)
        haug = jnp.concatenate([h, jnp.ones((1, T2), jnp.float32)], axis=0)
        delta = jnp.dot(w2_ref[...], haug,
                        preferred_element_type=jnp.float32)
        delta = jnp.tanh(delta) * (1.5 / 256.0)
        u = jnp.clip(bu_ref[...] + delta[0:1, :], 0.0, 1.0)
        v = jnp.clip(bv_ref[...] + delta[1:2, :], 0.0, 1.0)
        du = u * float(W_F - 1)
        dv = v * float(H_T - 1)
        j0f = jnp.floor(du)
        i0f = jnp.floor(dv)
        j0 = j0f.astype(jnp.int32)
        i0 = i0f.astype(jnp.int32)
        j1 = jnp.minimum(j0 + 1, W_F - 1)
        i1 = jnp.minimum(i0 + 1, H_T - 1)
        j1f = j1.astype(jnp.float32)
        i1f = i1.astype(jnp.float32)
        w00 = (i1f - dv) * (j1f - du)
        w10 = (i1f - dv) * (du - j0f)
        w01 = (dv - i0f) * (j1f - du)
        w11 = (dv - i0f) * (du - j0f)
        r0 = i0 * W_F
        r1 = i1 * W_F
        idx_ref[...] = (jnp.concatenate(
            [r0 + j0, r0 + j1, r1 + j0, r1 + j1], axis=0) + boff
        ).reshape(4, T2 // 128, 128)
        w_ref[...] = jnp.concatenate(
            [w00, w10, w01, w11], axis=0).reshape(4, T2 // 128, 128)

    return pl.pallas_call(
        body,
        grid=(Q // T2,),
        in_specs=[
            pl.BlockSpec((1, T2), lambda i: (0, i)),
            pl.BlockSpec((1, T2), lambda i: (0, i)),
            pl.BlockSpec((1, T2), lambda i: (0, i)),
            pl.BlockSpec((1, T2), lambda i: (0, i % nb)),
            pl.BlockSpec((1, T2), lambda i: (0, i % nb)),
            pl.BlockSpec((16, 24), lambda i: (0, 0)),
            pl.BlockSpec((16, 4 * NFREQ), lambda i: (0, 0)),
            pl.BlockSpec((2, 17), lambda i: (0, 0)),
        ],
        out_specs=[
            pl.BlockSpec((4, T2 // 128, 128), lambda i: (0, i, 0)),
            pl.BlockSpec((4, T2 // 128, 128), lambda i: (0, i, 0)),
        ],
        out_shape=[
            jax.ShapeDtypeStruct((4, Q // 128, 128), jnp.int32),
            jax.ShapeDtypeStruct((4, Q // 128, 128), jnp.float32),
        ],
    )(chrf, start, length, bu, bv, Achr, Wpos, Wj2a)


def _shuffle(vec, idx):
    return lax.gather(
        vec, idx.reshape(16, 1),
        lax.GatherDimensionNumbers(offset_dims=(), collapsed_slice_dims=(0,),
                                   start_index_map=(0,)),
        slice_sizes=(1,), mode=lax.GatherScatterMode.PROMISE_IN_BOUNDS)


def _lane_bcast(vec, p):
    return _shuffle(vec, jnp.full((16,), p, jnp.int32))


def _sc_combine(ytab_in, idx3, w3, bcu16, cpat, qpat):
    mesh = plsc.VectorSubcoreMesh(core_axis_name="c", subcore_axis_name="s")

    @functools.partial(
        pl.kernel,
        out_type=jax.ShapeDtypeStruct((Q // 16, 96), jnp.float32),
        mesh=mesh,
        compiler_params=pltpu.CompilerParams(use_tc_tiling_on_sc=False),
        scratch_types=[
            pltpu.VMEM((4, CCH // SUB, SUB), jnp.int32),
            pltpu.VMEM((4, CCH // SUB, SUB), jnp.float32),
            pltpu.VMEM((4, CCH, EP), jnp.float32),
            pltpu.VMEM((CCH // 16, 96), jnp.float32),
            pltpu.VMEM((16,), jnp.float32),
            pltpu.VMEM((6, 16), jnp.int32),
            pltpu.VMEM((6, 16), jnp.int32),
            pltpu.SemaphoreType.DMA,
        ],
    )
    def combine(y_h, idx_h, w_h, bcu_h, cpat_h, qpat_h, out_h,
                idx_v, w_v, rows_v, outc_v, bcu_v, cpat_v, qpat_v, sem):
        wid = lax.axis_index("s") * NC + lax.axis_index("c")
        pltpu.sync_copy(bcu_h, bcu_v)
        pltpu.sync_copy(cpat_h, cpat_v)
        pltpu.sync_copy(qpat_h, qpat_v)
        ytab = y_h
        qsets = [sorted(set((t * 16 + u) // 6 for u in range(16)))
                 for t in range(6)]

        def chunk(c, carry):
            base = pl.multiple_of(wid * QPW + c * CCH, CCH)
            rbase = pl.multiple_of(base // SUB, CCH // SUB)
            for k in range(4):
                pltpu.sync_copy(idx_h.at[k, pl.ds(rbase, CCH // SUB)],
                                idx_v.at[k])
                pltpu.sync_copy(w_h.at[k, pl.ds(rbase, CCH // SUB)],
                                w_v.at[k])
            cps = []
            for k in range(4):
                for j in range(CCH // SUB):
                    cps.append(pltpu.async_copy(
                        ytab.at[idx_v.at[k, j]],
                        rows_v.at[k, pl.ds(j * SUB, SUB)], sem))
            for cp in cps:
                cp.wait()
            bcu = bcu_v[...]

            def gstep(g, qcarry):
                qb = g * 16
                jrow = g // 8
                col = (g % 8) * 16
                w16 = [w_v[k, jrow, pl.ds(col, 16)] for k in range(4)]
                accs = []
                for p in range(16):
                    acc = bcu
                    for k in range(4):
                        acc = acc + rows_v[k, qb + p] * _lane_bcast(w16[k], p)
                    accs.append(acc)
                for t in range(6):
                    cvec = cpat_v[t]
                    qvec = qpat_v[t]
                    merged = None
                    for qk in qsets[t]:
                        sh = _shuffle(accs[qk], cvec)
                        merged = sh if merged is None else jnp.where(
                            qvec == qk, sh, merged)
                    outc_v[g, pl.ds(t * 16, 16)] = merged
                return qcarry

            lax.fori_loop(0, CCH // 16, gstep, 0)
            pltpu.sync_copy(
                outc_v,
                out_h.at[pl.ds(pl.multiple_of(base // 16, CCH // 16),
                               CCH // 16)])
            return carry

        lax.fori_loop(0, QPW // CCH, chunk, 0)

    return combine(ytab_in, idx3, w3, bcu16, cpat, qpat)


def kernel(z, spans, W_fl, b_fl, W_j1, b_j1, W_j2, b_j2, chr_table, W_cu,
           b_cu, base_coords):
    zp = jnp.swapaxes(z.reshape(B, C2, G), 1, 2).reshape(B * G // 8, 8 * C2)
    Wp = jnp.zeros((EP, C2), jnp.float32).at[:6, :].set(W_cu @ W_fl)
    Wbig = jnp.kron(jnp.eye(8, dtype=jnp.float32), Wp.T)
    bias16 = jnp.zeros((1, EP), jnp.float32).at[0, :6].set(W_cu @ b_fl)
    bias128 = jnp.tile(bias16, (1, 8))
    y128 = _stage1(zp, Wbig, bias128)

    chrf = spans[..., 0].reshape(1, Q)
    start = spans[..., 1].reshape(1, Q)
    length = spans[..., 2].reshape(1, Q)
    bu = base_coords[:, 0].reshape(1, N)
    bv = base_coords[:, 1].reshape(1, N)
    Achr = jnp.zeros((16, 24), jnp.float32).at[:, :23].set(
        (chr_table @ W_j1[:, :CHR_EMB].T + b_j1[None, :]).T)
    Wpos = W_j1[:, CHR_EMB:]
    Wj2a = jnp.concatenate([W_j2, b_j2[:, None]], axis=1)
    idxs, ws = _stage2(chrf, start, length, bu, bv, Achr, Wpos, Wj2a)

    bcu16 = jnp.zeros((16,), jnp.float32).at[:6].set(b_cu)
    cpat = jnp.asarray([[(t * 16 + u) % 6 for u in range(16)]
                        for t in range(6)], jnp.int32)
    qpat = jnp.asarray([[(t * 16 + u) // 6 for u in range(16)]
                        for t in range(6)], jnp.int32)
    out = _sc_combine(y128.reshape(B * G, EP), idxs, ws, bcu16, cpat, qpat)
    return out.reshape(B, N, 6)


# speedup vs baseline: 1.1560x; 1.1560x over previous
import functools
import math

import jax
import jax.numpy as jnp
from jax import lax
from jax.experimental import pallas as pl
from jax.experimental.pallas import tpu as pltpu
from jax.experimental.pallas import tpu_sc as plsc

B = 8
N = 65536
C2 = 32
H_T = 256
W_F = 256
G = H_T * W_F
Q = B * N
NFREQ = 8
CHR_EMB = 8
EP = 16
T1 = 2048
T2 = 2048
NC = 2
NS = 16
NW = NC * NS
QPW = Q // NW
CCH = 512
SUB = 128


def _stage1(zp, Wbig, bias128):
    TR = T1 // 8

    def body(z_ref, w_ref, b_ref, y_ref):
        y = jnp.dot(z_ref[...], w_ref[...],
                    preferred_element_type=jnp.float32)
        y_ref[...] = y + b_ref[...]

    return pl.pallas_call(
        body,
        grid=(B * G // T1,),
        in_specs=[
            pl.BlockSpec((TR, 8 * C2), lambda g: (g, 0)),
            pl.BlockSpec((8 * C2, 8 * EP), lambda g: (0, 0)),
            pl.BlockSpec((1, 8 * EP), lambda g: (0, 0)),
        ],
        out_specs=pl.BlockSpec((TR, 8 * EP), lambda g: (g, 0)),
        out_shape=jax.ShapeDtypeStruct((B * G // 8, 8 * EP), jnp.float32),
    )(zp, Wbig, bias128)


def _stage2(chrf, start, length, bu, bv, Achr, Wpos, Wj2a):
    nb = N // T2

    def body(c_ref, s_ref, l_ref, bu_ref, bv_ref, a_ref, wp_ref, w2_ref,
             idx_ref, w_ref):
        pid = pl.program_id(0)
        boff = (pid // nb) * G
        ci = c_ref[...].astype(jnp.int32)
        oh = (lax.broadcasted_iota(jnp.int32, (24, 1), 0) == ci
              ).astype(jnp.float32)
        fr = jnp.exp2(lax.broadcasted_iota(jnp.int32, (NFREQ, 1), 0)
                      .astype(jnp.float32))
        tp = 2.0 * math.pi
        s = tp * (fr * s_ref[...])
        l = tp * (fr * l_ref[...])
        pos = jnp.concatenate(
            [jnp.sin(s), jnp.cos(s), jnp.sin(l), jnp.cos(l)], axis=0)
        h = (jnp.dot(a_ref[...], oh, preferred_element_type=jnp.float32)
             + jnp.dot(wp_ref[...], pos, preferred_element_type=jnp.float32))
        h = jnp.where(h >= 0, h, 0.2 * h)
        haug = jnp.concatenate([h, jnp.ones((1, T2), jnp.float32)], axis=0)
        delta = jnp.dot(w2_ref[...], haug,
                        preferred_element_type=jnp.float32)
        delta = jnp.tanh(delta) * (1.5 / 256.0)
        u = jnp.clip(bu_ref[...] + delta[0:1, :], 0.0, 1.0)
        v = jnp.clip(bv_ref[...] + delta[1:2, :], 0.0, 1.0)
        du = u * float(W_F - 1)
        dv = v * float(H_T - 1)
        j0f = jnp.floor(du)
        i0f = jnp.floor(dv)
        j0 = j0f.astype(jnp.int32)
        i0 = i0f.astype(jnp.int32)
        j1 = jnp.minimum(j0 + 1, W_F - 1)
        i1 = jnp.minimum(i0 + 1, H_T - 1)
        j1f = j1.astype(jnp.float32)
        i1f = i1.astype(jnp.float32)
        w00 = (i1f - dv) * (j1f - du)
        w10 = (i1f - dv) * (du - j0f)
        w01 = (dv - i0f) * (j1f - du)
        w11 = (dv - i0f) * (du - j0f)
        r0 = i0 * W_F
        r1 = i1 * W_F
        idx_ref[...] = (jnp.concatenate(
            [r0 + j0, r0 + j1, r1 + j0, r1 + j1], axis=0) + boff
        ).reshape(4, T2 // 128, 128)
        w_ref[...] = jnp.concatenate(
            [w00, w10, w01, w11], axis=0).reshape(4, T2 // 128, 128)

    return pl.pallas_call(
        body,
        grid=(Q // T2,),
        in_specs=[
            pl.BlockSpec((1, T2), lambda i: (0, i)),
            pl.BlockSpec((1, T2), lambda i: (0, i)),
            pl.BlockSpec((1, T2), lambda i: (0, i)),
            pl.BlockSpec((1, T2), lambda i: (0, i % nb)),
            pl.BlockSpec((1, T2), lambda i: (0, i % nb)),
            pl.BlockSpec((16, 24), lambda i: (0, 0)),
            pl.BlockSpec((16, 4 * NFREQ), lambda i: (0, 0)),
            pl.BlockSpec((2, 17), lambda i: (0, 0)),
        ],
        out_specs=[
            pl.BlockSpec((4, T2 // 128, 128), lambda i: (0, i, 0)),
            pl.BlockSpec((4, T2 // 128, 128), lambda i: (0, i, 0)),
        ],
        out_shape=[
            jax.ShapeDtypeStruct((4, Q // 128, 128), jnp.int32),
            jax.ShapeDtypeStruct((4, Q // 128, 128), jnp.float32),
        ],
    )(chrf, start, length, bu, bv, Achr, Wpos, Wj2a)


def _shuffle(vec, idx):
    return lax.gather(
        vec, idx.reshape(16, 1),
        lax.GatherDimensionNumbers(offset_dims=(), collapsed_slice_dims=(0,),
                                   start_index_map=(0,)),
        slice_sizes=(1,), mode=lax.GatherScatterMode.PROMISE_IN_BOUNDS)


def _lane_bcast(vec, p):
    return _shuffle(vec, jnp.full((16,), p, jnp.int32))


def _sc_combine(ytab_in, idx3, w3, bcu16):
    mesh = plsc.VectorSubcoreMesh(core_axis_name="c", subcore_axis_name="s")
    NSUB = CCH // SUB
    NCH = QPW // CCH

    @functools.partial(
        pl.kernel,
        out_type=jax.ShapeDtypeStruct((Q // 8, 8 * EP), jnp.float32),
        mesh=mesh,
        compiler_params=pltpu.CompilerParams(use_tc_tiling_on_sc=False),
        scratch_types=[
            [pltpu.VMEM((4, NSUB, SUB), jnp.int32) for _ in range(2)],
            [pltpu.VMEM((4, NSUB, SUB), jnp.float32) for _ in range(2)],
            [pltpu.VMEM((4, CCH, EP), jnp.float32) for _ in range(2)],
            pltpu.VMEM((CCH // 8, 8 * EP), jnp.float32),
            pltpu.VMEM((16,), jnp.float32),
            [pltpu.SemaphoreType.DMA for _ in range(2)],
        ],
    )
    def combine(y_h, idx_h, w_h, bcu_h, out_h,
                idx_v, w_v, rows_v, out_v, bcu_v, sems):
        wid = lax.axis_index("s") * NC + lax.axis_index("c")
        pltpu.sync_copy(bcu_h, bcu_v)

        def stage_and_fire(c, buf):
            base = pl.multiple_of(wid * QPW + c * CCH, CCH)
            rbase = pl.multiple_of(base // SUB, NSUB)
            for k in range(4):
                pltpu.sync_copy(idx_h.at[k, pl.ds(rbase, NSUB)],
                                idx_v[buf].at[k])
                pltpu.sync_copy(w_h.at[k, pl.ds(rbase, NSUB)],
                                w_v[buf].at[k])
            for k in range(4):
                for j in range(NSUB):
                    pltpu.async_copy(y_h.at[idx_v[buf].at[k, j]],
                                     rows_v[buf].at[k, pl.ds(j * SUB, SUB)],
                                     sems[buf])

        def drain(buf):
            for k in range(4):
                for j in range(NSUB):
                    pltpu.make_async_copy(
                        y_h.at[idx_v[buf].at[k, j]],
                        rows_v[buf].at[k, pl.ds(j * SUB, SUB)],
                        sems[buf]).wait()

        def do_combine(c, buf):
            base = pl.multiple_of(wid * QPW + c * CCH, CCH)
            bcu = bcu_v[...]

            def gstep(g, qcarry):
                qb = g * 16
                jrow = g // 8
                col = (g % 8) * 16
                w16 = [w_v[buf][k, jrow, pl.ds(col, 16)] for k in range(4)]
                for p in range(16):
                    acc = bcu
                    for k in range(4):
                        acc = acc + (rows_v[buf][k, qb + p]
                                     * _lane_bcast(w16[k], p))
                    out_v[g * 2 + p // 8, pl.ds((p % 8) * EP, EP)] = acc
                return qcarry

            lax.fori_loop(0, CCH // 16, gstep, 0)
            pltpu.sync_copy(
                out_v,
                out_h.at[pl.ds(pl.multiple_of(base // 8, CCH // 8),
                               CCH // 8)])

        stage_and_fire(0, 0)

        def step(c2, carry):
            c = c2 * 2
            stage_and_fire(c + 1, 1)
            drain(0)
            do_combine(c, 0)

            @pl.when(c + 2 < NCH)
            def _():
                stage_and_fire(c + 2, 0)

            drain(1)
            do_combine(c + 1, 1)
            return carry

        lax.fori_loop(0, NCH // 2, step, 0)

    return combine(ytab_in, idx3, w3, bcu16)


def kernel(z, spans, W_fl, b_fl, W_j1, b_j1, W_j2, b_j2, chr_table, W_cu,
           b_cu, base_coords):
    zp = jnp.swapaxes(z.reshape(B, C2, G), 1, 2).reshape(B * G // 8, 8 * C2)
    Wp = jnp.zeros((EP, C2), jnp.float32).at[:6, :].set(W_cu @ W_fl)
    Wbig = jnp.kron(jnp.eye(8, dtype=jnp.float32), Wp.T)
    bias16 = jnp.zeros((1, EP), jnp.float32).at[0, :6].set(W_cu @ b_fl)
    bias128 = jnp.tile(bias16, (1, 8))
    y128 = _stage1(zp, Wbig, bias128)

    chrf = spans[..., 0].reshape(1, Q)
    start = spans[..., 1].reshape(1, Q)
    length = spans[..., 2].reshape(1, Q)
    bu = base_coords[:, 0].reshape(1, N)
    bv = base_coords[:, 1].reshape(1, N)
    Achr = jnp.zeros((16, 24), jnp.float32).at[:, :23].set(
        (chr_table @ W_j1[:, :CHR_EMB].T + b_j1[None, :]).T)
    Wpos = W_j1[:, CHR_EMB:]
    Wj2a = jnp.concatenate([W_j2, b_j2[:, None]], axis=1)
    idxs, ws = _stage2(chrf, start, length, bu, bv, Achr, Wpos, Wj2a)

    bcu16 = jnp.zeros((16,), jnp.float32).at[:6].set(b_cu)
    out = _sc_combine(y128.reshape(B * G, EP), idxs, ws, bcu16)
    return out.reshape(Q, EP)[:, :6].reshape(B, N, 6)


# speedup vs baseline: 1.1987x; 1.0370x over previous
import functools
import math

import jax
import jax.numpy as jnp
from jax import lax
from jax.experimental import pallas as pl
from jax.experimental.pallas import tpu as pltpu
from jax.experimental.pallas import tpu_sc as plsc

B = 8
N = 65536
C2 = 32
H_T = 256
W_F = 256
G = H_T * W_F
Q = B * N
NFREQ = 8
CHR_EMB = 8
EP = 16
T1 = 2048
T2 = 2048
NC = 2
NS = 16
NW = NC * NS
QPW = Q // NW
CCH = 512
SUB = 128


def _stage1(zp, Wbig, bias128):
    TR = T1 // 8

    def body(z_ref, w_ref, b_ref, y_ref):
        y = jnp.dot(z_ref[...], w_ref[...],
                    preferred_element_type=jnp.float32)
        y_ref[...] = y + b_ref[...]

    return pl.pallas_call(
        body,
        grid=(B * G // T1,),
        in_specs=[
            pl.BlockSpec((TR, 8 * C2), lambda g: (g, 0)),
            pl.BlockSpec((8 * C2, 8 * EP), lambda g: (0, 0)),
            pl.BlockSpec((1, 8 * EP), lambda g: (0, 0)),
        ],
        out_specs=pl.BlockSpec((TR, 8 * EP), lambda g: (g, 0)),
        out_shape=jax.ShapeDtypeStruct((B * G // 8, 8 * EP), jnp.float32),
    )(zp, Wbig, bias128)


def _sincos_octaves(x):
    ang = (2.0 * math.pi) * x
    s, c = jnp.sin(ang), jnp.cos(ang)
    sins, coss = [s], [c]
    for _ in range(NFREQ - 1):
        s, c = 2.0 * s * c, 1.0 - 2.0 * s * s
        sins.append(s)
        coss.append(c)
    return sins, coss


def _stage2(spans_t, bc_t, Achr, Wpos, Wj2a):
    nb = N // T2

    def body(sp_ref, bc_ref, a_ref, wp_ref, w2_ref, idx_ref, w_ref):
        pid = pl.program_id(0)
        boff = (pid // nb) * G
        ci = sp_ref[0:1, :].astype(jnp.int32)
        oh = (lax.broadcasted_iota(jnp.int32, (24, 1), 0) == ci
              ).astype(jnp.float32)
        ss, cs = _sincos_octaves(sp_ref[1:2, :])
        sl, cl = _sincos_octaves(sp_ref[2:3, :])
        pos = jnp.concatenate(ss + cs + sl + cl, axis=0)
        h = (jnp.dot(a_ref[...], oh, preferred_element_type=jnp.float32)
             + jnp.dot(wp_ref[...], pos, preferred_element_type=jnp.float32))
        h = jnp.where(h >= 0, h, 0.2 * h)
        haug = jnp.concatenate([h, jnp.ones((1, T2), jnp.float32)], axis=0)
        delta = jnp.dot(w2_ref[...], haug,
                        preferred_element_type=jnp.float32)
        delta = jnp.tanh(delta) * (1.5 / 256.0)
        u = jnp.clip(bc_ref[0:1, :] + delta[0:1, :], 0.0, 1.0)
        v = jnp.clip(bc_ref[1:2, :] + delta[1:2, :], 0.0, 1.0)
        du = u * float(W_F - 1)
        dv = v * float(H_T - 1)
        j0f = jnp.floor(du)
        i0f = jnp.floor(dv)
        j0 = j0f.astype(jnp.int32)
        i0 = i0f.astype(jnp.int32)
        j1 = jnp.minimum(j0 + 1, W_F - 1)
        i1 = jnp.minimum(i0 + 1, H_T - 1)
        j1f = j1.astype(jnp.float32)
        i1f = i1.astype(jnp.float32)
        w00 = (i1f - dv) * (j1f - du)
        w10 = (i1f - dv) * (du - j0f)
        w01 = (dv - i0f) * (j1f - du)
        w11 = (dv - i0f) * (du - j0f)
        r0 = i0 * W_F
        r1 = i1 * W_F
        idx_ref[...] = (jnp.concatenate(
            [r0 + j0, r0 + j1, r1 + j0, r1 + j1], axis=0) + boff
        ).reshape(4, T2 // 128, 128)
        w_ref[...] = jnp.concatenate(
            [w00, w10, w01, w11], axis=0).reshape(4, T2 // 128, 128)

    return pl.pallas_call(
        body,
        grid=(Q // T2,),
        in_specs=[
            pl.BlockSpec((3, T2), lambda i: (0, i)),
            pl.BlockSpec((2, T2), lambda i: (0, i % nb)),
            pl.BlockSpec((16, 24), lambda i: (0, 0)),
            pl.BlockSpec((16, 4 * NFREQ), lambda i: (0, 0)),
            pl.BlockSpec((2, 17), lambda i: (0, 0)),
        ],
        out_specs=[
            pl.BlockSpec((4, T2 // 128, 128), lambda i: (0, i, 0)),
            pl.BlockSpec((4, T2 // 128, 128), lambda i: (0, i, 0)),
        ],
        out_shape=[
            jax.ShapeDtypeStruct((4, Q // 128, 128), jnp.int32),
            jax.ShapeDtypeStruct((4, Q // 128, 128), jnp.float32),
        ],
    )(spans_t, bc_t, Achr, Wpos, Wj2a)


def _shuffle(vec, idx):
    return lax.gather(
        vec, idx.reshape(16, 1),
        lax.GatherDimensionNumbers(offset_dims=(), collapsed_slice_dims=(0,),
                                   start_index_map=(0,)),
        slice_sizes=(1,), mode=lax.GatherScatterMode.PROMISE_IN_BOUNDS)


def _lane_bcast(vec, p):
    return _shuffle(vec, jnp.full((16,), p, jnp.int32))


def _sc_combine(ytab_in, idx3, w3, bcu16):
    mesh = plsc.VectorSubcoreMesh(core_axis_name="c", subcore_axis_name="s")
    NSUB = CCH // SUB
    NCH = QPW // CCH

    @functools.partial(
        pl.kernel,
        out_type=jax.ShapeDtypeStruct((Q // 8, 8 * EP), jnp.float32),
        mesh=mesh,
        compiler_params=pltpu.CompilerParams(use_tc_tiling_on_sc=False),
        scratch_types=[
            [pltpu.VMEM((4, NSUB, SUB), jnp.int32) for _ in range(2)],
            [pltpu.VMEM((4, NSUB, SUB), jnp.float32) for _ in range(2)],
            [pltpu.VMEM((4, CCH, EP), jnp.float32) for _ in range(2)],
            pltpu.VMEM((CCH // 8, 8 * EP), jnp.float32),
            pltpu.VMEM((16,), jnp.float32),
            [pltpu.SemaphoreType.DMA for _ in range(2)],
        ],
    )
    def combine(y_h, idx_h, w_h, bcu_h, out_h,
                idx_v, w_v, rows_v, out_v, bcu_v, sems):
        wid = lax.axis_index("s") * NC + lax.axis_index("c")
        pltpu.sync_copy(bcu_h, bcu_v)

        def stage_and_fire(c, buf):
            base = pl.multiple_of(wid * QPW + c * CCH, CCH)
            rbase = pl.multiple_of(base // SUB, NSUB)
            for k in range(4):
                pltpu.sync_copy(idx_h.at[k, pl.ds(rbase, NSUB)],
                                idx_v[buf].at[k])
                pltpu.sync_copy(w_h.at[k, pl.ds(rbase, NSUB)],
                                w_v[buf].at[k])
            for k in range(4):
                for j in range(NSUB):
                    pltpu.async_copy(y_h.at[idx_v[buf].at[k, j]],
                                     rows_v[buf].at[k, pl.ds(j * SUB, SUB)],
                                     sems[buf])

        def drain(buf):
            for k in range(4):
                for j in range(NSUB):
                    pltpu.make_async_copy(
                        y_h.at[idx_v[buf].at[k, j]],
                        rows_v[buf].at[k, pl.ds(j * SUB, SUB)],
                        sems[buf]).wait()

        def do_combine(c, buf):
            base = pl.multiple_of(wid * QPW + c * CCH, CCH)
            bcu = bcu_v[...]

            def gstep(g, qcarry):
                qb = g * 16
                jrow = g // 8
                col = (g % 8) * 16
                w16 = [w_v[buf][k, jrow, pl.ds(col, 16)] for k in range(4)]
                for p in range(16):
                    acc = bcu
                    for k in range(4):
                        acc = acc + (rows_v[buf][k, qb + p]
                                     * _lane_bcast(w16[k], p))
                    out_v[g * 2 + p // 8, pl.ds((p % 8) * EP, EP)] = acc
                return qcarry

            lax.fori_loop(0, CCH // 16, gstep, 0)
            pltpu.sync_copy(
                out_v,
                out_h.at[pl.ds(pl.multiple_of(base // 8, CCH // 8),
                               CCH // 8)])

        stage_and_fire(0, 0)

        def step(c2, carry):
            c = c2 * 2
            stage_and_fire(c + 1, 1)
            drain(0)
            do_combine(c, 0)

            @pl.when(c + 2 < NCH)
            def _():
                stage_and_fire(c + 2, 0)

            drain(1)
            do_combine(c + 1, 1)
            return carry

        lax.fori_loop(0, NCH // 2, step, 0)

    return combine(ytab_in, idx3, w3, bcu16)


def kernel(z, spans, W_fl, b_fl, W_j1, b_j1, W_j2, b_j2, chr_table, W_cu,
           b_cu, base_coords):
    zp = jnp.swapaxes(z.reshape(B, C2, G), 1, 2).reshape(B * G // 8, 8 * C2)
    Wp = jnp.zeros((EP, C2), jnp.float32).at[:6, :].set(W_cu @ W_fl)
    Wbig = jnp.kron(jnp.eye(8, dtype=jnp.float32), Wp.T)
    bias16 = jnp.zeros((1, EP), jnp.float32).at[0, :6].set(W_cu @ b_fl)
    bias128 = jnp.tile(bias16, (1, 8))
    y128 = _stage1(zp, Wbig, bias128)

    spans_t = jnp.swapaxes(spans.reshape(Q, 3), 0, 1)
    bc_t = jnp.swapaxes(base_coords, 0, 1)
    Achr = jnp.zeros((16, 24), jnp.float32).at[:, :23].set(
        (chr_table @ W_j1[:, :CHR_EMB].T + b_j1[None, :]).T)
    Wpos = W_j1[:, CHR_EMB:]
    Wj2a = jnp.concatenate([W_j2, b_j2[:, None]], axis=1)
    idxs, ws = _stage2(spans_t, bc_t, Achr, Wpos, Wj2a)

    bcu16 = jnp.zeros((16,), jnp.float32).at[:6].set(b_cu)
    out = _sc_combine(y128.reshape(B * G, EP), idxs, ws, bcu16)
    return out.reshape(Q, EP)[:, :6].reshape(B, N, 6)


# speedup vs baseline: 1.2937x; 1.0792x over previous
import functools
import math

import jax
import jax.numpy as jnp
from jax import lax
from jax.experimental import pallas as pl
from jax.experimental.pallas import tpu as pltpu
from jax.experimental.pallas import tpu_sc as plsc

B = 8
N = 65536
C2 = 32
H_T = 256
W_F = 256
G = H_T * W_F
Q = B * N
NFREQ = 8
CHR_EMB = 8
EP = 16
T1 = 2048
T2 = 4096
NC = 2
NS = 16
NW = NC * NS
QPW = Q // NW
CCH = 512
SUB = 128


def _stage1(zp, Wbig, bias128):
    TR = T1 // 8

    def body(z_ref, w_ref, b_ref, y_ref):
        y = jnp.dot(z_ref[...], w_ref[...],
                    preferred_element_type=jnp.float32)
        y_ref[...] = y + b_ref[...]

    return pl.pallas_call(
        body,
        grid=(B * G // T1,),
        in_specs=[
            pl.BlockSpec((TR, 8 * C2), lambda g: (g, 0)),
            pl.BlockSpec((8 * C2, 8 * EP), lambda g: (0, 0)),
            pl.BlockSpec((1, 8 * EP), lambda g: (0, 0)),
        ],
        out_specs=pl.BlockSpec((TR, 8 * EP), lambda g: (g, 0)),
        out_shape=jax.ShapeDtypeStruct((B * G // 8, 8 * EP), jnp.float32),
    )(zp, Wbig, bias128)


def _sincos_octaves(x):
    ang = (2.0 * math.pi) * x
    s, c = jnp.sin(ang), jnp.cos(ang)
    sins, coss = [s], [c]
    for _ in range(NFREQ - 1):
        s, c = 2.0 * s * c, 1.0 - 2.0 * s * s
        sins.append(s)
        coss.append(c)
    return sins, coss


def _stage2(spans_t, bc_t, Achr, Wpos, Wj2a):
    nb = N // T2

    def body(sp_ref, bc_ref, a_ref, wp_ref, w2_ref, idx_ref, w_ref):
        pid = pl.program_id(0)
        boff = (pid // nb) * G
        ci = sp_ref[0:1, :].astype(jnp.int32)
        oh = (lax.broadcasted_iota(jnp.int32, (24, 1), 0) == ci
              ).astype(jnp.float32)
        ss, cs = _sincos_octaves(sp_ref[1:2, :])
        sl, cl = _sincos_octaves(sp_ref[2:3, :])
        pos = jnp.concatenate(ss + cs + sl + cl, axis=0)
        h = (jnp.dot(a_ref[...], oh, preferred_element_type=jnp.float32)
             + jnp.dot(wp_ref[...], pos, preferred_element_type=jnp.float32))
        h = jnp.where(h >= 0, h, 0.2 * h)
        haug = jnp.concatenate([h, jnp.ones((1, T2), jnp.float32)], axis=0)
        delta = jnp.dot(w2_ref[...], haug,
                        preferred_element_type=jnp.float32)
        delta = jnp.tanh(delta) * (1.5 / 256.0)
        u = jnp.clip(bc_ref[0:1, :] + delta[0:1, :], 0.0, 1.0)
        v = jnp.clip(bc_ref[1:2, :] + delta[1:2, :], 0.0, 1.0)
        du = u * float(W_F - 1)
        dv = v * float(H_T - 1)
        j0f = jnp.floor(du)
        i0f = jnp.floor(dv)
        j0 = j0f.astype(jnp.int32)
        i0 = i0f.astype(jnp.int32)
        j1 = jnp.minimum(j0 + 1, W_F - 1)
        i1 = jnp.minimum(i0 + 1, H_T - 1)
        j1f = j1.astype(jnp.float32)
        i1f = i1.astype(jnp.float32)
        w00 = (i1f - dv) * (j1f - du)
        w10 = (i1f - dv) * (du - j0f)
        w01 = (dv - i0f) * (j1f - du)
        w11 = (dv - i0f) * (du - j0f)
        r0 = i0 * W_F
        r1 = i1 * W_F
        idx_ref[...] = (jnp.concatenate(
            [r0 + j0, r0 + j1, r1 + j0, r1 + j1], axis=0) + boff
        ).reshape(4, T2 // 128, 128)
        w_ref[...] = jnp.concatenate(
            [w00, w10, w01, w11], axis=0).reshape(4, T2 // 128, 128)

    return pl.pallas_call(
        body,
        grid=(Q // T2,),
        in_specs=[
            pl.BlockSpec((3, T2), lambda i: (0, i)),
            pl.BlockSpec((2, T2), lambda i: (0, i % nb)),
            pl.BlockSpec((16, 24), lambda i: (0, 0)),
            pl.BlockSpec((16, 4 * NFREQ), lambda i: (0, 0)),
            pl.BlockSpec((2, 17), lambda i: (0, 0)),
        ],
        out_specs=[
            pl.BlockSpec((4, T2 // 128, 128), lambda i: (0, i, 0)),
            pl.BlockSpec((4, T2 // 128, 128), lambda i: (0, i, 0)),
        ],
        out_shape=[
            jax.ShapeDtypeStruct((4, Q // 128, 128), jnp.int32),
            jax.ShapeDtypeStruct((4, Q // 128, 128), jnp.float32),
        ],
    )(spans_t, bc_t, Achr, Wpos, Wj2a)


def _shuffle(vec, idx):
    return lax.gather(
        vec, idx.reshape(16, 1),
        lax.GatherDimensionNumbers(offset_dims=(), collapsed_slice_dims=(0,),
                                   start_index_map=(0,)),
        slice_sizes=(1,), mode=lax.GatherScatterMode.PROMISE_IN_BOUNDS)


def _lane_bcast(vec, p):
    return _shuffle(vec, jnp.full((16,), p, jnp.int32))


def _sc_combine(ytab_in, idx3, w3, bcu16):
    mesh = plsc.VectorSubcoreMesh(core_axis_name="c", subcore_axis_name="s")
    NSUB = CCH // SUB
    NCH = QPW // CCH

    @functools.partial(
        pl.kernel,
        out_type=jax.ShapeDtypeStruct((Q // 8, 8 * EP), jnp.float32),
        mesh=mesh,
        compiler_params=pltpu.CompilerParams(use_tc_tiling_on_sc=False),
        scratch_types=[
            [pltpu.VMEM((4, NSUB, SUB), jnp.int32) for _ in range(2)],
            [pltpu.VMEM((4, NSUB, SUB), jnp.float32) for _ in range(2)],
            [pltpu.VMEM((4, CCH, EP), jnp.float32) for _ in range(2)],
            pltpu.VMEM((CCH // 8, 8 * EP), jnp.float32),
            pltpu.VMEM((16,), jnp.float32),
            [pltpu.SemaphoreType.DMA for _ in range(2)],
        ],
    )
    def combine(y_h, idx_h, w_h, bcu_h, out_h,
                idx_v, w_v, rows_v, out_v, bcu_v, sems):
        wid = lax.axis_index("s") * NC + lax.axis_index("c")
        pltpu.sync_copy(bcu_h, bcu_v)

        def stage_and_fire(c, buf):
            base = pl.multiple_of(wid * QPW + c * CCH, CCH)
            rbase = pl.multiple_of(base // SUB, NSUB)
            for k in range(4):
                pltpu.sync_copy(idx_h.at[k, pl.ds(rbase, NSUB)],
                                idx_v[buf].at[k])
                pltpu.sync_copy(w_h.at[k, pl.ds(rbase, NSUB)],
                                w_v[buf].at[k])
            for k in range(4):
                for j in range(NSUB):
                    pltpu.async_copy(y_h.at[idx_v[buf].at[k, j]],
                                     rows_v[buf].at[k, pl.ds(j * SUB, SUB)],
                                     sems[buf])

        def drain(buf):
            for k in range(4):
                for j in range(NSUB):
                    pltpu.make_async_copy(
                        y_h.at[idx_v[buf].at[k, j]],
                        rows_v[buf].at[k, pl.ds(j * SUB, SUB)],
                        sems[buf]).wait()

        def do_combine(c, buf):
            base = pl.multiple_of(wid * QPW + c * CCH, CCH)
            bcu = bcu_v[...]

            def gstep(g, qcarry):
                qb = g * 16
                jrow = g // 8
                col = (g % 8) * 16
                w16 = [w_v[buf][k, jrow, pl.ds(col, 16)] for k in range(4)]
                for p in range(16):
                    acc = bcu
                    for k in range(4):
                        acc = acc + (rows_v[buf][k, qb + p]
                                     * _lane_bcast(w16[k], p))
                    out_v[g * 2 + p // 8, pl.ds((p % 8) * EP, EP)] = acc
                return qcarry

            lax.fori_loop(0, CCH // 16, gstep, 0, unroll=2)
            pltpu.sync_copy(
                out_v,
                out_h.at[pl.ds(pl.multiple_of(base // 8, CCH // 8),
                               CCH // 8)])

        stage_and_fire(0, 0)

        def step(c2, carry):
            c = c2 * 2
            stage_and_fire(c + 1, 1)
            drain(0)
            do_combine(c, 0)

            @pl.when(c + 2 < NCH)
            def _():
                stage_and_fire(c + 2, 0)

            drain(1)
            do_combine(c + 1, 1)
            return carry

        lax.fori_loop(0, NCH // 2, step, 0)

    return combine(ytab_in, idx3, w3, bcu16)


def kernel(z, spans, W_fl, b_fl, W_j1, b_j1, W_j2, b_j2, chr_table, W_cu,
           b_cu, base_coords):
    zp = jnp.swapaxes(z.reshape(B, C2, G), 1, 2).reshape(B * G // 8, 8 * C2)
    Wp = jnp.zeros((EP, C2), jnp.float32).at[:6, :].set(W_cu @ W_fl)
    Wbig = jnp.kron(jnp.eye(8, dtype=jnp.float32), Wp.T)
    bias16 = jnp.zeros((1, EP), jnp.float32).at[0, :6].set(W_cu @ b_fl)
    bias128 = jnp.tile(bias16, (1, 8))
    y128 = _stage1(zp, Wbig, bias128)

    spans_t = jnp.swapaxes(spans.reshape(Q, 3), 0, 1)
    bc_t = jnp.swapaxes(base_coords, 0, 1)
    Achr = jnp.zeros((16, 24), jnp.float32).at[:, :23].set(
        (chr_table @ W_j1[:, :CHR_EMB].T + b_j1[None, :]).T)
    Wpos = W_j1[:, CHR_EMB:]
    Wj2a = jnp.concatenate([W_j2, b_j2[:, None]], axis=1)
    idxs, ws = _stage2(spans_t, bc_t, Achr, Wpos, Wj2a)

    bcu16 = jnp.zeros((16,), jnp.float32).at[:6].set(b_cu)
    out = _sc_combine(y128.reshape(B * G, EP), idxs, ws, bcu16)
    return out.reshape(Q, EP)[:, :6].reshape(B, N, 6)


# speedup vs baseline: 1.4593x; 1.1280x over previous
import functools
import math

import jax
import jax.numpy as jnp
from jax import lax
from jax.experimental import pallas as pl
from jax.experimental.pallas import tpu as pltpu
from jax.experimental.pallas import tpu_sc as plsc

B = 8
N = 65536
C2 = 32
H_T = 256
W_F = 256
G = H_T * W_F
Q = B * N
NFREQ = 8
CHR_EMB = 8
EP = 16
T1 = 4096
T2 = 8192
NC = 2
NS = 16
NW = NC * NS
QPW = Q // NW
CCH = 512
SUB = 128


def _stage1(zp, Wbig, bias128):
    TR = T1 // 8

    def body(z_ref, w_ref, b_ref, y_ref):
        y = jnp.dot(z_ref[...], w_ref[...],
                    preferred_element_type=jnp.float32)
        y_ref[...] = y + b_ref[...]

    return pl.pallas_call(
        body,
        grid=(B * G // T1,),
        in_specs=[
            pl.BlockSpec((TR, 8 * C2), lambda g: (g, 0)),
            pl.BlockSpec((8 * C2, 8 * EP), lambda g: (0, 0)),
            pl.BlockSpec((1, 8 * EP), lambda g: (0, 0)),
        ],
        out_specs=pl.BlockSpec((TR, 8 * EP), lambda g: (g, 0)),
        out_shape=jax.ShapeDtypeStruct((B * G // 8, 8 * EP), jnp.float32),
    )(zp, Wbig, bias128)


def _sincos_octaves(x):
    ang = (2.0 * math.pi) * x
    s, c = jnp.sin(ang), jnp.cos(ang)
    sins, coss = [s], [c]
    for _ in range(NFREQ - 1):
        s, c = 2.0 * s * c, 1.0 - 2.0 * s * s
        sins.append(s)
        coss.append(c)
    return sins, coss


def _stage2(spans_t, bc_t, Achr, Wpos, Wj2a):
    nb = N // T2

    def body(sp_ref, bc_ref, a_ref, wp_ref, w2_ref, idx_ref, w_ref):
        pid = pl.program_id(0)
        boff = (pid // nb) * G
        ci = sp_ref[0:1, :].astype(jnp.int32)
        oh = (lax.broadcasted_iota(jnp.int32, (24, 1), 0) == ci
              ).astype(jnp.float32)
        ss, cs = _sincos_octaves(sp_ref[1:2, :])
        sl, cl = _sincos_octaves(sp_ref[2:3, :])
        pos = jnp.concatenate(ss + cs + sl + cl, axis=0)
        h = (jnp.dot(a_ref[...], oh, preferred_element_type=jnp.float32)
             + jnp.dot(wp_ref[...], pos, preferred_element_type=jnp.float32))
        h = jnp.where(h >= 0, h, 0.2 * h)
        haug = jnp.concatenate([h, jnp.ones((1, T2), jnp.float32)], axis=0)
        delta = jnp.dot(w2_ref[...], haug,
                        preferred_element_type=jnp.float32)
        delta = jnp.tanh(delta) * (1.5 / 256.0)
        u = jnp.clip(bc_ref[0:1, :] + delta[0:1, :], 0.0, 1.0)
        v = jnp.clip(bc_ref[1:2, :] + delta[1:2, :], 0.0, 1.0)
        du = u * float(W_F - 1)
        dv = v * float(H_T - 1)
        j0f = jnp.floor(du)
        i0f = jnp.floor(dv)
        j0 = j0f.astype(jnp.int32)
        i0 = i0f.astype(jnp.int32)
        j1 = jnp.minimum(j0 + 1, W_F - 1)
        i1 = jnp.minimum(i0 + 1, H_T - 1)
        j1f = j1.astype(jnp.float32)
        i1f = i1.astype(jnp.float32)
        w00 = (i1f - dv) * (j1f - du)
        w10 = (i1f - dv) * (du - j0f)
        w01 = (dv - i0f) * (j1f - du)
        w11 = (dv - i0f) * (du - j0f)
        r0 = i0 * W_F
        r1 = i1 * W_F
        idx_ref[...] = (jnp.concatenate(
            [r0 + j0, r0 + j1, r1 + j0, r1 + j1], axis=0) + boff
        ).reshape(4, T2 // 128, 128)
        w_ref[...] = jnp.concatenate(
            [w00, w10, w01, w11], axis=0).reshape(4, T2 // 128, 128)

    return pl.pallas_call(
        body,
        grid=(Q // T2,),
        in_specs=[
            pl.BlockSpec((3, T2), lambda i: (0, i)),
            pl.BlockSpec((2, T2), lambda i: (0, i % nb)),
            pl.BlockSpec((16, 24), lambda i: (0, 0)),
            pl.BlockSpec((16, 4 * NFREQ), lambda i: (0, 0)),
            pl.BlockSpec((2, 17), lambda i: (0, 0)),
        ],
        out_specs=[
            pl.BlockSpec((4, T2 // 128, 128), lambda i: (0, i, 0)),
            pl.BlockSpec((4, T2 // 128, 128), lambda i: (0, i, 0)),
        ],
        out_shape=[
            jax.ShapeDtypeStruct((4, Q // 128, 128), jnp.int32),
            jax.ShapeDtypeStruct((4, Q // 128, 128), jnp.float32),
        ],
    )(spans_t, bc_t, Achr, Wpos, Wj2a)


def _shuffle(vec, idx):
    return lax.gather(
        vec, idx.reshape(16, 1),
        lax.GatherDimensionNumbers(offset_dims=(), collapsed_slice_dims=(0,),
                                   start_index_map=(0,)),
        slice_sizes=(1,), mode=lax.GatherScatterMode.PROMISE_IN_BOUNDS)


def _lane_bcast(vec, p):
    return _shuffle(vec, jnp.full((16,), p, jnp.int32))


def _sc_combine(ytab_in, idx3, w3, bcu16):
    mesh = plsc.VectorSubcoreMesh(core_axis_name="c", subcore_axis_name="s")
    NSUB = CCH // SUB
    NCH = QPW // CCH

    @functools.partial(
        pl.kernel,
        out_type=jax.ShapeDtypeStruct((Q // 8, 8 * EP), jnp.float32),
        mesh=mesh,
        compiler_params=pltpu.CompilerParams(use_tc_tiling_on_sc=False),
        scratch_types=[
            [pltpu.VMEM((4, NSUB, SUB), jnp.int32) for _ in range(2)],
            [pltpu.VMEM((4, NSUB, SUB), jnp.float32) for _ in range(2)],
            [pltpu.VMEM((4, CCH, EP), jnp.float32) for _ in range(2)],
            pltpu.VMEM((CCH // 8, 8 * EP), jnp.float32),
            pltpu.VMEM((16,), jnp.float32),
            [pltpu.SemaphoreType.DMA for _ in range(2)],
        ],
    )
    def combine(y_h, idx_h, w_h, bcu_h, out_h,
                idx_v, w_v, rows_v, out_v, bcu_v, sems):
        wid = lax.axis_index("s") * NC + lax.axis_index("c")
        pltpu.sync_copy(bcu_h, bcu_v)

        def stage_and_fire(c, buf):
            base = pl.multiple_of(wid * QPW + c * CCH, CCH)
            rbase = pl.multiple_of(base // SUB, NSUB)
            for k in range(4):
                pltpu.sync_copy(idx_h.at[k, pl.ds(rbase, NSUB)],
                                idx_v[buf].at[k])
                pltpu.sync_copy(w_h.at[k, pl.ds(rbase, NSUB)],
                                w_v[buf].at[k])
            for k in range(4):
                for j in range(NSUB):
                    pltpu.async_copy(y_h.at[idx_v[buf].at[k, j]],
                                     rows_v[buf].at[k, pl.ds(j * SUB, SUB)],
                                     sems[buf])

        def drain(buf):
            for k in range(4):
                for j in range(NSUB):
                    pltpu.make_async_copy(
                        y_h.at[idx_v[buf].at[k, j]],
                        rows_v[buf].at[k, pl.ds(j * SUB, SUB)],
                        sems[buf]).wait()

        def do_combine(c, buf):
            base = pl.multiple_of(wid * QPW + c * CCH, CCH)
            bcu = bcu_v[...]

            def gstep(g, qcarry):
                qb = g * 16
                jrow = g // 8
                col = (g % 8) * 16
                w16 = [w_v[buf][k, jrow, pl.ds(col, 16)] for k in range(4)]
                for p in range(16):
                    acc = bcu
                    for k in range(4):
                        acc = acc + (rows_v[buf][k, qb + p]
                                     * _lane_bcast(w16[k], p))
                    out_v[g * 2 + p // 8, pl.ds((p % 8) * EP, EP)] = acc
                return qcarry

            lax.fori_loop(0, CCH // 16, gstep, 0, unroll=2)
            pltpu.sync_copy(
                out_v,
                out_h.at[pl.ds(pl.multiple_of(base // 8, CCH // 8),
                               CCH // 8)])

        stage_and_fire(0, 0)

        def step(c2, carry):
            c = c2 * 2
            stage_and_fire(c + 1, 1)
            drain(0)
            do_combine(c, 0)

            @pl.when(c + 2 < NCH)
            def _():
                stage_and_fire(c + 2, 0)

            drain(1)
            do_combine(c + 1, 1)
            return carry

        lax.fori_loop(0, NCH // 2, step, 0)

    return combine(ytab_in, idx3, w3, bcu16)


def kernel(z, spans, W_fl, b_fl, W_j1, b_j1, W_j2, b_j2, chr_table, W_cu,
           b_cu, base_coords):
    zp = jnp.swapaxes(z.reshape(B, C2, G), 1, 2).reshape(B * G // 8, 8 * C2)
    Wp = jnp.zeros((EP, C2), jnp.float32).at[:6, :].set(W_cu @ W_fl)
    Wbig = jnp.kron(jnp.eye(8, dtype=jnp.float32), Wp.T)
    bias16 = jnp.zeros((1, EP), jnp.float32).at[0, :6].set(W_cu @ b_fl)
    bias128 = jnp.tile(bias16, (1, 8))
    y128 = _stage1(zp, Wbig, bias128)

    spans_t = jnp.swapaxes(spans.reshape(Q, 3), 0, 1)
    bc_t = jnp.swapaxes(base_coords, 0, 1)
    Achr = jnp.zeros((16, 24), jnp.float32).at[:, :23].set(
        (chr_table @ W_j1[:, :CHR_EMB].T + b_j1[None, :]).T)
    Wpos = W_j1[:, CHR_EMB:]
    Wj2a = jnp.concatenate([W_j2, b_j2[:, None]], axis=1)
    idxs, ws = _stage2(spans_t, bc_t, Achr, Wpos, Wj2a)

    bcu16 = jnp.zeros((16,), jnp.float32).at[:6].set(b_cu)
    out = _sc_combine(y128.reshape(B * G, EP), idxs, ws, bcu16)
    return out.reshape(Q, EP)[:, :6].reshape(B, N, 6)


# speedup vs baseline: 1.5272x; 1.0465x over previous
import functools
import math

import jax
import jax.numpy as jnp
from jax import lax
from jax.experimental import pallas as pl
from jax.experimental.pallas import tpu as pltpu
from jax.experimental.pallas import tpu_sc as plsc

B = 8
N = 65536
C2 = 32
H_T = 256
W_F = 256
G = H_T * W_F
Q = B * N
NFREQ = 8
CHR_EMB = 8
EP = 16
T1 = 8192
T2 = 16384
NC = 2
NS = 16
NW = NC * NS
QPW = Q // NW
CCH = 512
SUB = 128


def _stage1(zp, Wbig, bias128):
    TR = T1 // 8

    def body(z_ref, w_ref, b_ref, y_ref):
        y = jnp.dot(z_ref[...], w_ref[...],
                    preferred_element_type=jnp.float32)
        y_ref[...] = y + b_ref[...]

    return pl.pallas_call(
        body,
        grid=(B * G // T1,),
        in_specs=[
            pl.BlockSpec((TR, 8 * C2), lambda g: (g, 0)),
            pl.BlockSpec((8 * C2, 8 * EP), lambda g: (0, 0)),
            pl.BlockSpec((1, 8 * EP), lambda g: (0, 0)),
        ],
        out_specs=pl.BlockSpec((TR, 8 * EP), lambda g: (g, 0)),
        out_shape=jax.ShapeDtypeStruct((B * G // 8, 8 * EP), jnp.float32),
    )(zp, Wbig, bias128)


def _sincos_octaves(x):
    ang = (2.0 * math.pi) * x
    s, c = jnp.sin(ang), jnp.cos(ang)
    sins, coss = [s], [c]
    for _ in range(NFREQ - 1):
        s, c = 2.0 * s * c, 1.0 - 2.0 * s * s
        sins.append(s)
        coss.append(c)
    return sins, coss


def _stage2(spans_t, bc_t, Achr, Wpos, Wj2a):
    nb = N // T2

    def body(sp_ref, bc_ref, a_ref, wp_ref, w2_ref, idx_ref, w_ref):
        pid = pl.program_id(0)
        boff = (pid // nb) * G
        ci = sp_ref[0:1, :].astype(jnp.int32)
        oh = (lax.broadcasted_iota(jnp.int32, (24, 1), 0) == ci
              ).astype(jnp.float32)
        ss, cs = _sincos_octaves(sp_ref[1:2, :])
        sl, cl = _sincos_octaves(sp_ref[2:3, :])
        pos = jnp.concatenate(ss + cs + sl + cl, axis=0)
        h = (jnp.dot(a_ref[...], oh, preferred_element_type=jnp.float32)
             + jnp.dot(wp_ref[...], pos, preferred_element_type=jnp.float32))
        h = jnp.where(h >= 0, h, 0.2 * h)
        haug = jnp.concatenate([h, jnp.ones((1, T2), jnp.float32)], axis=0)
        delta = jnp.dot(w2_ref[...], haug,
                        preferred_element_type=jnp.float32)
        delta = jnp.tanh(delta) * (1.5 / 256.0)
        u = jnp.clip(bc_ref[0:1, :] + delta[0:1, :], 0.0, 1.0)
        v = jnp.clip(bc_ref[1:2, :] + delta[1:2, :], 0.0, 1.0)
        du = u * float(W_F - 1)
        dv = v * float(H_T - 1)
        j0f = jnp.floor(du)
        i0f = jnp.floor(dv)
        j0 = j0f.astype(jnp.int32)
        i0 = i0f.astype(jnp.int32)
        j1 = jnp.minimum(j0 + 1, W_F - 1)
        i1 = jnp.minimum(i0 + 1, H_T - 1)
        j1f = j1.astype(jnp.float32)
        i1f = i1.astype(jnp.float32)
        w00 = (i1f - dv) * (j1f - du)
        w10 = (i1f - dv) * (du - j0f)
        w01 = (dv - i0f) * (j1f - du)
        w11 = (dv - i0f) * (du - j0f)
        r0 = i0 * W_F
        r1 = i1 * W_F
        idx_ref[...] = (jnp.concatenate(
            [r0 + j0, r0 + j1, r1 + j0, r1 + j1], axis=0) + boff
        ).reshape(4, T2 // 128, 128)
        w_ref[...] = jnp.concatenate(
            [w00, w10, w01, w11], axis=0).reshape(4, T2 // 128, 128)

    return pl.pallas_call(
        body,
        grid=(Q // T2,),
        in_specs=[
            pl.BlockSpec((3, T2), lambda i: (0, i)),
            pl.BlockSpec((2, T2), lambda i: (0, i % nb)),
            pl.BlockSpec((16, 24), lambda i: (0, 0)),
            pl.BlockSpec((16, 4 * NFREQ), lambda i: (0, 0)),
            pl.BlockSpec((2, 17), lambda i: (0, 0)),
        ],
        out_specs=[
            pl.BlockSpec((4, T2 // 128, 128), lambda i: (0, i, 0)),
            pl.BlockSpec((4, T2 // 128, 128), lambda i: (0, i, 0)),
        ],
        out_shape=[
            jax.ShapeDtypeStruct((4, Q // 128, 128), jnp.int32),
            jax.ShapeDtypeStruct((4, Q // 128, 128), jnp.float32),
        ],
    )(spans_t, bc_t, Achr, Wpos, Wj2a)


def _shuffle(vec, idx):
    return lax.gather(
        vec, idx.reshape(16, 1),
        lax.GatherDimensionNumbers(offset_dims=(), collapsed_slice_dims=(0,),
                                   start_index_map=(0,)),
        slice_sizes=(1,), mode=lax.GatherScatterMode.PROMISE_IN_BOUNDS)


def _lane_bcast(vec, p):
    return _shuffle(vec, jnp.full((16,), p, jnp.int32))


def _sc_combine(ytab_in, idx3, w3, bcu16):
    mesh = plsc.VectorSubcoreMesh(core_axis_name="c", subcore_axis_name="s")
    NSUB = CCH // SUB
    NCH = QPW // CCH

    @functools.partial(
        pl.kernel,
        out_type=jax.ShapeDtypeStruct((Q // 8, 8 * EP), jnp.float32),
        mesh=mesh,
        compiler_params=pltpu.CompilerParams(use_tc_tiling_on_sc=False),
        scratch_types=[
            [pltpu.VMEM((4, NSUB, SUB), jnp.int32) for _ in range(2)],
            [pltpu.VMEM((4, NSUB, SUB), jnp.float32) for _ in range(2)],
            [pltpu.VMEM((4, CCH, EP), jnp.float32) for _ in range(2)],
            pltpu.VMEM((CCH // 8, 8 * EP), jnp.float32),
            pltpu.VMEM((16,), jnp.float32),
            [pltpu.SemaphoreType.DMA for _ in range(2)],
        ],
    )
    def combine(y_h, idx_h, w_h, bcu_h, out_h,
                idx_v, w_v, rows_v, out_v, bcu_v, sems):
        wid = lax.axis_index("s") * NC + lax.axis_index("c")
        pltpu.sync_copy(bcu_h, bcu_v)

        def stage_and_fire(c, buf):
            base = pl.multiple_of(wid * QPW + c * CCH, CCH)
            rbase = pl.multiple_of(base // SUB, NSUB)
            for k in range(4):
                pltpu.sync_copy(idx_h.at[k, pl.ds(rbase, NSUB)],
                                idx_v[buf].at[k])
                pltpu.sync_copy(w_h.at[k, pl.ds(rbase, NSUB)],
                                w_v[buf].at[k])
            for k in range(4):
                for j in range(NSUB):
                    pltpu.async_copy(y_h.at[idx_v[buf].at[k, j]],
                                     rows_v[buf].at[k, pl.ds(j * SUB, SUB)],
                                     sems[buf])

        def drain(buf):
            for k in range(4):
                for j in range(NSUB):
                    pltpu.make_async_copy(
                        y_h.at[idx_v[buf].at[k, j]],
                        rows_v[buf].at[k, pl.ds(j * SUB, SUB)],
                        sems[buf]).wait()

        def do_combine(c, buf):
            base = pl.multiple_of(wid * QPW + c * CCH, CCH)
            bcu = bcu_v[...]

            def gstep(g, qcarry):
                qb = g * 16
                jrow = g // 8
                col = (g % 8) * 16
                w16 = [w_v[buf][k, jrow, pl.ds(col, 16)] for k in range(4)]
                for p in range(16):
                    acc = bcu
                    for k in range(4):
                        acc = acc + (rows_v[buf][k, qb + p]
                                     * _lane_bcast(w16[k], p))
                    out_v[g * 2 + p // 8, pl.ds((p % 8) * EP, EP)] = acc
                return qcarry

            lax.fori_loop(0, CCH // 16, gstep, 0, unroll=2)
            pltpu.sync_copy(
                out_v,
                out_h.at[pl.ds(pl.multiple_of(base // 8, CCH // 8),
                               CCH // 8)])

        stage_and_fire(0, 0)

        def step(c2, carry):
            c = c2 * 2
            stage_and_fire(c + 1, 1)
            drain(0)
            do_combine(c, 0)

            @pl.when(c + 2 < NCH)
            def _():
                stage_and_fire(c + 2, 0)

            drain(1)
            do_combine(c + 1, 1)
            return carry

        lax.fori_loop(0, NCH // 2, step, 0)

    return combine(ytab_in, idx3, w3, bcu16)


def kernel(z, spans, W_fl, b_fl, W_j1, b_j1, W_j2, b_j2, chr_table, W_cu,
           b_cu, base_coords):
    zp = jnp.swapaxes(z.reshape(B, C2, G), 1, 2).reshape(B * G // 8, 8 * C2)
    Wp = jnp.zeros((EP, C2), jnp.float32).at[:6, :].set(W_cu @ W_fl)
    Wbig = jnp.kron(jnp.eye(8, dtype=jnp.float32), Wp.T)
    bias16 = jnp.zeros((1, EP), jnp.float32).at[0, :6].set(W_cu @ b_fl)
    bias128 = jnp.tile(bias16, (1, 8))
    y128 = _stage1(zp, Wbig, bias128)

    spans_t = jnp.swapaxes(spans.reshape(Q, 3), 0, 1)
    bc_t = jnp.swapaxes(base_coords, 0, 1)
    Achr = jnp.zeros((16, 24), jnp.float32).at[:, :23].set(
        (chr_table @ W_j1[:, :CHR_EMB].T + b_j1[None, :]).T)
    Wpos = W_j1[:, CHR_EMB:]
    Wj2a = jnp.concatenate([W_j2, b_j2[:, None]], axis=1)
    idxs, ws = _stage2(spans_t, bc_t, Achr, Wpos, Wj2a)

    bcu16 = jnp.zeros((16,), jnp.float32).at[:6].set(b_cu)
    out = _sc_combine(y128.reshape(B * G, EP), idxs, ws, bcu16)
    return out.reshape(Q, EP)[:, :6].reshape(B, N, 6)


# speedup vs baseline: 1.5600x; 1.0215x over previous
import functools
import math

import jax
import jax.numpy as jnp
from jax import lax
from jax.experimental import pallas as pl
from jax.experimental.pallas import tpu as pltpu
from jax.experimental.pallas import tpu_sc as plsc

B = 8
N = 65536
C2 = 32
H_T = 256
W_F = 256
G = H_T * W_F
Q = B * N
NFREQ = 8
CHR_EMB = 8
EP = 16
T1 = 16384
T2 = 32768
NC = 2
NS = 16
NW = NC * NS
QPW = Q // NW
CCH = 512
SUB = 128


def _stage1(zp, Wbig, bias128):
    TR = T1 // 8

    def body(z_ref, w_ref, b_ref, y_ref):
        y = jnp.dot(z_ref[...], w_ref[...],
                    preferred_element_type=jnp.float32)
        y_ref[...] = y + b_ref[...]

    return pl.pallas_call(
        body,
        grid=(B * G // T1,),
        in_specs=[
            pl.BlockSpec((TR, 8 * C2), lambda g: (g, 0)),
            pl.BlockSpec((8 * C2, 8 * EP), lambda g: (0, 0)),
            pl.BlockSpec((1, 8 * EP), lambda g: (0, 0)),
        ],
        out_specs=pl.BlockSpec((TR, 8 * EP), lambda g: (g, 0)),
        out_shape=jax.ShapeDtypeStruct((B * G // 8, 8 * EP), jnp.float32),
    )(zp, Wbig, bias128)


def _sincos_octaves(x):
    ang = (2.0 * math.pi) * x
    s, c = jnp.sin(ang), jnp.cos(ang)
    sins, coss = [s], [c]
    for _ in range(NFREQ - 1):
        s, c = 2.0 * s * c, 1.0 - 2.0 * s * s
        sins.append(s)
        coss.append(c)
    return sins, coss


def _stage2(spans_t, bc_t, Achr, Wpos, Wj2a):
    nb = N // T2

    def body(sp_ref, bc_ref, a_ref, wp_ref, w2_ref, idx_ref, w_ref):
        pid = pl.program_id(0)
        boff = (pid // nb) * G
        ci = sp_ref[0:1, :].astype(jnp.int32)
        oh = (lax.broadcasted_iota(jnp.int32, (24, 1), 0) == ci
              ).astype(jnp.float32)
        ss, cs = _sincos_octaves(sp_ref[1:2, :])
        sl, cl = _sincos_octaves(sp_ref[2:3, :])
        pos = jnp.concatenate(ss + cs + sl + cl, axis=0)
        h = (jnp.dot(a_ref[...], oh, preferred_element_type=jnp.float32)
             + jnp.dot(wp_ref[...], pos, preferred_element_type=jnp.float32))
        h = jnp.where(h >= 0, h, 0.2 * h)
        haug = jnp.concatenate([h, jnp.ones((1, T2), jnp.float32)], axis=0)
        delta = jnp.dot(w2_ref[...], haug,
                        preferred_element_type=jnp.float32)
        delta = jnp.tanh(delta) * (1.5 / 256.0)
        u = jnp.clip(bc_ref[0:1, :] + delta[0:1, :], 0.0, 1.0)
        v = jnp.clip(bc_ref[1:2, :] + delta[1:2, :], 0.0, 1.0)
        du = u * float(W_F - 1)
        dv = v * float(H_T - 1)
        j0f = jnp.floor(du)
        i0f = jnp.floor(dv)
        j0 = j0f.astype(jnp.int32)
        i0 = i0f.astype(jnp.int32)
        j1 = jnp.minimum(j0 + 1, W_F - 1)
        i1 = jnp.minimum(i0 + 1, H_T - 1)
        j1f = j1.astype(jnp.float32)
        i1f = i1.astype(jnp.float32)
        w00 = (i1f - dv) * (j1f - du)
        w10 = (i1f - dv) * (du - j0f)
        w01 = (dv - i0f) * (j1f - du)
        w11 = (dv - i0f) * (du - j0f)
        r0 = i0 * W_F
        r1 = i1 * W_F
        idx_ref[...] = (jnp.concatenate(
            [r0 + j0, r0 + j1, r1 + j0, r1 + j1], axis=0) + boff
        ).reshape(4, T2 // 128, 128)
        w_ref[...] = jnp.concatenate(
            [w00, w10, w01, w11], axis=0).reshape(4, T2 // 128, 128)

    return pl.pallas_call(
        body,
        grid=(Q // T2,),
        in_specs=[
            pl.BlockSpec((3, T2), lambda i: (0, i)),
            pl.BlockSpec((2, T2), lambda i: (0, i % nb)),
            pl.BlockSpec((16, 24), lambda i: (0, 0)),
            pl.BlockSpec((16, 4 * NFREQ), lambda i: (0, 0)),
            pl.BlockSpec((2, 17), lambda i: (0, 0)),
        ],
        out_specs=[
            pl.BlockSpec((4, T2 // 128, 128), lambda i: (0, i, 0)),
            pl.BlockSpec((4, T2 // 128, 128), lambda i: (0, i, 0)),
        ],
        out_shape=[
            jax.ShapeDtypeStruct((4, Q // 128, 128), jnp.int32),
            jax.ShapeDtypeStruct((4, Q // 128, 128), jnp.float32),
        ],
    )(spans_t, bc_t, Achr, Wpos, Wj2a)


def _shuffle(vec, idx):
    return lax.gather(
        vec, idx.reshape(16, 1),
        lax.GatherDimensionNumbers(offset_dims=(), collapsed_slice_dims=(0,),
                                   start_index_map=(0,)),
        slice_sizes=(1,), mode=lax.GatherScatterMode.PROMISE_IN_BOUNDS)


def _lane_bcast(vec, p):
    return _shuffle(vec, jnp.full((16,), p, jnp.int32))


def _sc_combine(ytab_in, idx3, w3, bcu16):
    mesh = plsc.VectorSubcoreMesh(core_axis_name="c", subcore_axis_name="s")
    NSUB = CCH // SUB
    NCH = QPW // CCH

    @functools.partial(
        pl.kernel,
        out_type=jax.ShapeDtypeStruct((Q // 8, 8 * EP), jnp.float32),
        mesh=mesh,
        compiler_params=pltpu.CompilerParams(use_tc_tiling_on_sc=False),
        scratch_types=[
            [pltpu.VMEM((4, NSUB, SUB), jnp.int32) for _ in range(2)],
            [pltpu.VMEM((4, NSUB, SUB), jnp.float32) for _ in range(2)],
            [pltpu.VMEM((4, CCH, EP), jnp.float32) for _ in range(2)],
            pltpu.VMEM((CCH // 8, 8 * EP), jnp.float32),
            pltpu.VMEM((16,), jnp.float32),
            [pltpu.SemaphoreType.DMA for _ in range(2)],
        ],
    )
    def combine(y_h, idx_h, w_h, bcu_h, out_h,
                idx_v, w_v, rows_v, out_v, bcu_v, sems):
        wid = lax.axis_index("s") * NC + lax.axis_index("c")
        pltpu.sync_copy(bcu_h, bcu_v)

        def stage_and_fire(c, buf):
            base = pl.multiple_of(wid * QPW + c * CCH, CCH)
            rbase = pl.multiple_of(base // SUB, NSUB)
            for k in range(4):
                pltpu.sync_copy(idx_h.at[k, pl.ds(rbase, NSUB)],
                                idx_v[buf].at[k])
                pltpu.sync_copy(w_h.at[k, pl.ds(rbase, NSUB)],
                                w_v[buf].at[k])
            for k in range(4):
                for j in range(NSUB):
                    pltpu.async_copy(y_h.at[idx_v[buf].at[k, j]],
                                     rows_v[buf].at[k, pl.ds(j * SUB, SUB)],
                                     sems[buf])

        def drain(buf):
            for k in range(4):
                for j in range(NSUB):
                    pltpu.make_async_copy(
                        y_h.at[idx_v[buf].at[k, j]],
                        rows_v[buf].at[k, pl.ds(j * SUB, SUB)],
                        sems[buf]).wait()

        def do_combine(c, buf):
            base = pl.multiple_of(wid * QPW + c * CCH, CCH)
            bcu = bcu_v[...]

            def gstep(g, qcarry):
                qb = g * 16
                jrow = g // 8
                col = (g % 8) * 16
                w16 = [w_v[buf][k, jrow, pl.ds(col, 16)] for k in range(4)]
                for p in range(16):
                    acc = bcu
                    for k in range(4):
                        acc = acc + (rows_v[buf][k, qb + p]
                                     * _lane_bcast(w16[k], p))
                    out_v[g * 2 + p // 8, pl.ds((p % 8) * EP, EP)] = acc
                return qcarry

            lax.fori_loop(0, CCH // 16, gstep, 0, unroll=2)
            pltpu.sync_copy(
                out_v,
                out_h.at[pl.ds(pl.multiple_of(base // 8, CCH // 8),
                               CCH // 8)])

        stage_and_fire(0, 0)

        def step(c2, carry):
            c = c2 * 2
            stage_and_fire(c + 1, 1)
            drain(0)
            do_combine(c, 0)

            @pl.when(c + 2 < NCH)
            def _():
                stage_and_fire(c + 2, 0)

            drain(1)
            do_combine(c + 1, 1)
            return carry

        lax.fori_loop(0, NCH // 2, step, 0)

    return combine(ytab_in, idx3, w3, bcu16)


def kernel(z, spans, W_fl, b_fl, W_j1, b_j1, W_j2, b_j2, chr_table, W_cu,
           b_cu, base_coords):
    zp = jnp.swapaxes(z.reshape(B, C2, G), 1, 2).reshape(B * G // 8, 8 * C2)
    Wp = jnp.zeros((EP, C2), jnp.float32).at[:6, :].set(W_cu @ W_fl)
    Wbig = jnp.kron(jnp.eye(8, dtype=jnp.float32), Wp.T)
    bias16 = jnp.zeros((1, EP), jnp.float32).at[0, :6].set(W_cu @ b_fl)
    bias128 = jnp.tile(bias16, (1, 8))
    y128 = _stage1(zp, Wbig, bias128)

    spans_t = jnp.swapaxes(spans.reshape(Q, 3), 0, 1)
    bc_t = jnp.swapaxes(base_coords, 0, 1)
    Achr = jnp.zeros((16, 24), jnp.float32).at[:, :23].set(
        (chr_table @ W_j1[:, :CHR_EMB].T + b_j1[None, :]).T)
    Wpos = W_j1[:, CHR_EMB:]
    Wj2a = jnp.concatenate([W_j2, b_j2[:, None]], axis=1)
    idxs, ws = _stage2(spans_t, bc_t, Achr, Wpos, Wj2a)

    bcu16 = jnp.zeros((16,), jnp.float32).at[:6].set(b_cu)
    out = _sc_combine(y128.reshape(B * G, EP), idxs, ws, bcu16)
    return out.reshape(Q, EP)[:, :6].reshape(B, N, 6)
